# Initial kernel scaffold; baseline (speedup 1.0000x reference)
import jax
import jax.numpy as jnp
from jax.experimental import pallas as pl


def kernel(x, gate_w, gate_b, expert_w, expert_b):
    raise NotImplementedError("write your pallas kernel here")


# speedup vs baseline: 12.9082x; 12.9082x over previous
import functools

import jax
import jax.numpy as jnp
from jax.experimental import pallas as pl
from jax.experimental.pallas import tpu as pltpu

_C = 8
_E = 8
_OUT = 8
_NEG = -3.0e38


def _tc_body(gw_ref, gb_ref, ew_ref, eb_ref, x_ref, final_ref, logits_ref):
    xs = [x_ref[0, c] for c in range(_C)]

    xb = [v.astype(jnp.bfloat16).astype(jnp.float32) for v in xs]
    ls = []
    for c in range(_C):
        acc = jnp.full_like(xs[0], gb_ref[0, c])
        for k in range(_C):
            gwk = gw_ref[c, k].astype(jnp.bfloat16).astype(jnp.float32)
            acc = acc + gwk * xb[k]
        ls.append(acc)
        logits_ref[0, c] = acc

    m1 = ls[0]
    for c in range(1, _C):
        m1 = jnp.maximum(m1, ls[c])
    t1 = []
    found = None
    for c in range(_C):
        eq = ls[c] == m1
        if found is None:
            t1.append(eq)
            found = eq
        else:
            t1.append(eq & (~found))
            found = found | eq
    masked = [jnp.where(t1[c], _NEG, ls[c]) for c in range(_C)]
    m2 = masked[0]
    for c in range(1, _C):
        m2 = jnp.maximum(m2, masked[c])
    t2 = []
    found = None
    for c in range(_C):
        eq = masked[c] == m2
        if found is None:
            t2.append(eq)
            found = eq
        else:
            t2.append(eq & (~found))
            found = found | eq

    w2 = 1.0 / (1.0 + jnp.exp(m1 - m2))
    w1 = 1.0 - w2
    zero = jnp.zeros_like(w1)
    ce = [jnp.where(t1[c], w1, jnp.where(t2[c], w2, zero)) for c in range(_C)]

    fin = [None] * _OUT
    for e in range(_E):
        for o in range(_OUT):
            y = jnp.full_like(xs[0], eb_ref[e, o])
            for k in range(_C):
                y = y + ew_ref[e * _OUT + o, k] * xs[k]
            contrib = ce[e] * y
            fin[o] = contrib if fin[o] is None else fin[o] + contrib
    for o in range(_OUT):
        final_ref[0, o] = fin[o]


@functools.partial(jax.jit, static_argnames=("hb",))
def _run_tc(x, gw, gb, ew, eb, hb=32):
    B, C, H, W = x.shape
    grid = (B, H // hb)
    smem = functools.partial(pl.BlockSpec, memory_space=pltpu.SMEM)
    out_shape = [
        jax.ShapeDtypeStruct((B, _OUT, H, W), x.dtype),
        jax.ShapeDtypeStruct((B, C, H, W), jnp.float32),
    ]
    f = pl.pallas_call(
        _tc_body,
        grid=grid,
        in_specs=[
            smem((C, C), lambda b, h: (0, 0)),
            smem((1, C), lambda b, h: (0, 0)),
            smem((_E * _OUT, C), lambda b, h: (0, 0)),
            smem((_E, _OUT), lambda b, h: (0, 0)),
            pl.BlockSpec((1, C, hb, W), lambda b, h: (b, 0, h, 0)),
        ],
        out_specs=[
            pl.BlockSpec((1, _OUT, hb, W), lambda b, h: (b, 0, h, 0)),
            pl.BlockSpec((1, C, hb, W), lambda b, h: (b, 0, h, 0)),
        ],
        out_shape=out_shape,
    )
    return f(gw, gb, ew, eb, x)


def kernel(x, gate_w, gate_b, expert_w, expert_b):
    gw = gate_w.reshape(_C, _C)
    gb = gate_b.reshape(1, _C)
    ew = expert_w.reshape(_E * _OUT, _C)
    eb = expert_b.reshape(_E, _OUT)
    final, logits = _run_tc(x, gw, gb, ew, eb)
    return (final, logits)


# speedup vs baseline: 16.0414x; 1.2427x over previous
import functools

import jax
import jax.numpy as jnp
from jax.experimental import pallas as pl
from jax.experimental.pallas import tpu as pltpu

_C = 8
_E = 8
_OUT = 8
_NEG = -3.0e38


def _tc_body(gw_ref, gb_ref, ew_ref, eb_ref, x_ref, final_ref, logits_ref):
    xs = [x_ref[0, c] for c in range(_C)]

    xb = [v.astype(jnp.bfloat16).astype(jnp.float32) for v in xs]
    ls = []
    for c in range(_C):
        acc = jnp.full_like(xs[0], gb_ref[0, c])
        for k in range(_C):
            gwk = gw_ref[c, k].astype(jnp.bfloat16).astype(jnp.float32)
            acc = acc + gwk * xb[k]
        ls.append(acc)
        logits_ref[0, c] = acc

    m1 = ls[0]
    for c in range(1, _C):
        m1 = jnp.maximum(m1, ls[c])
    t1 = []
    found = None
    for c in range(_C):
        eq = ls[c] == m1
        if found is None:
            t1.append(eq)
            found = eq
        else:
            t1.append(eq & (~found))
            found = found | eq
    masked = [jnp.where(t1[c], _NEG, ls[c]) for c in range(_C)]
    m2 = masked[0]
    for c in range(1, _C):
        m2 = jnp.maximum(m2, masked[c])
    t2 = []
    found = None
    for c in range(_C):
        eq = masked[c] == m2
        if found is None:
            t2.append(eq)
            found = eq
        else:
            t2.append(eq & (~found))
            found = found | eq

    w2 = 1.0 / (1.0 + jnp.exp(m1 - m2))
    w1 = 1.0 - w2
    zero = jnp.zeros_like(w1)
    ce = [jnp.where(t1[c], w1, jnp.where(t2[c], w2, zero)) for c in range(_C)]

    xp = [v.astype(jnp.bfloat16) for v in xs]
    fin = [None] * _OUT
    for e in range(_E):
        for o in range(_OUT):
            y = ew_ref[e * _OUT + o, 0].astype(jnp.bfloat16) * xp[0]
            for k in range(1, _C):
                y = y + ew_ref[e * _OUT + o, k].astype(jnp.bfloat16) * xp[k]
            y = y + eb_ref[e, o].astype(jnp.bfloat16)
            contrib = ce[e] * y.astype(jnp.float32)
            fin[o] = contrib if fin[o] is None else fin[o] + contrib
    for o in range(_OUT):
        final_ref[0, o] = fin[o]


@functools.partial(jax.jit, static_argnames=("hb",))
def _run_tc(x, gw, gb, ew, eb, hb=32):
    B, C, H, W = x.shape
    grid = (B, H // hb)
    smem = functools.partial(pl.BlockSpec, memory_space=pltpu.SMEM)
    out_shape = [
        jax.ShapeDtypeStruct((B, _OUT, H, W), x.dtype),
        jax.ShapeDtypeStruct((B, C, H, W), jnp.float32),
    ]
    f = pl.pallas_call(
        _tc_body,
        grid=grid,
        in_specs=[
            smem((C, C), lambda b, h: (0, 0)),
            smem((1, C), lambda b, h: (0, 0)),
            smem((_E * _OUT, C), lambda b, h: (0, 0)),
            smem((_E, _OUT), lambda b, h: (0, 0)),
            pl.BlockSpec((1, C, hb, W), lambda b, h: (b, 0, h, 0)),
        ],
        out_specs=[
            pl.BlockSpec((1, _OUT, hb, W), lambda b, h: (b, 0, h, 0)),
            pl.BlockSpec((1, C, hb, W), lambda b, h: (b, 0, h, 0)),
        ],
        out_shape=out_shape,
    )
    return f(gw, gb, ew, eb, x)


def kernel(x, gate_w, gate_b, expert_w, expert_b):
    gw = gate_w.reshape(_C, _C)
    gb = gate_b.reshape(1, _C)
    ew = expert_w.reshape(_E * _OUT, _C)
    eb = expert_b.reshape(_E, _OUT)
    final, logits = _run_tc(x, gw, gb, ew, eb)
    return (final, logits)
